# initial kernel scaffold (unmeasured)
import functools

import jax
import jax.numpy as jnp
from jax import lax
from jax.experimental import pallas as pl
from jax.experimental.pallas import tpu as pltpu

N_DEV = 4


def _layer(x, win, wout, collective_id):
    b, d = x.shape
    _, h_per = win.shape

    def body(x_ref, win_ref, wout_ref, out_ref, comm_ref, send_sems, recv_sems):
        my = lax.axis_index("i")
        left = (my - 1) % N_DEV
        right = (my + 1) % N_DEV

        barrier_sem = pltpu.get_barrier_semaphore()
        for nbr in (left, right):
            pl.semaphore_signal(
                barrier_sem, inc=1,
                device_id=(nbr,), device_id_type=pl.DeviceIdType.MESH,
            )
        pl.semaphore_wait(barrier_sem, 2)

        h = jnp.dot(x_ref[:, :], win_ref[:, :], preferred_element_type=jnp.float32)
        h = jnp.maximum(h, 0.0)
        partial = jnp.dot(h, wout_ref[:, :], preferred_element_type=jnp.float32)

        out_ref[:, :] = partial
        comm_ref[0, :, :] = partial

        for hop in range(N_DEV - 1):
            rdma = pltpu.make_async_remote_copy(
                src_ref=comm_ref.at[hop],
                dst_ref=comm_ref.at[hop + 1],
                send_sem=send_sems.at[hop],
                recv_sem=recv_sems.at[hop],
                device_id=(right,),
                device_id_type=pl.DeviceIdType.MESH,
            )
            rdma.start()
            rdma.wait()
            out_ref[:, :] += comm_ref[hop + 1, :, :]

    return pl.pallas_call(
        body,
        out_shape=jax.ShapeDtypeStruct((b, d), jnp.float32),
        in_specs=[
            pl.BlockSpec(memory_space=pltpu.VMEM),
            pl.BlockSpec(memory_space=pltpu.VMEM),
            pl.BlockSpec(memory_space=pltpu.VMEM),
        ],
        out_specs=pl.BlockSpec(memory_space=pltpu.VMEM),
        scratch_shapes=[
            pltpu.VMEM((N_DEV, b, d), jnp.float32),
            pltpu.SemaphoreType.DMA((N_DEV - 1,)),
            pltpu.SemaphoreType.DMA((N_DEV - 1,)),
        ],
        compiler_params=pltpu.CompilerParams(collective_id=collective_id),
    )(x, win, wout)


def kernel(x, Win0, Wout0, Win1, Wout1, Win2, Wout2):
    x = _layer(x, Win0, Wout0, collective_id=0)
    x = _layer(x, Win1, Wout1, collective_id=1)
    x = _layer(x, Win2, Wout2, collective_id=2)
    return x


# baseline (device time: 147070 ns/iter reference)
import jax
import jax.numpy as jnp
from jax import lax
from jax.experimental import pallas as pl
from jax.experimental.pallas import tpu as pltpu

N_DEV = 4
N_CHUNKS = 4


def _layer(x, win, wout, collective_id):
    b, d = x.shape
    _, h_per = win.shape
    chunk = h_per // N_CHUNKS

    def body(x_ref, win_ref, wout_ref, out_ref, comm_ref, send_sems, recv_sems):
        k = pl.program_id(0)

        h = jnp.dot(x_ref[:, :], win_ref[:, :], preferred_element_type=jnp.float32)
        h = jnp.maximum(h, 0.0)
        partial = jnp.dot(h, wout_ref[:, :], preferred_element_type=jnp.float32)

        @pl.when(k == 0)
        def _():
            out_ref[:, :] = partial

        @pl.when(k != 0)
        def _():
            out_ref[:, :] += partial

        @pl.when(k == N_CHUNKS - 1)
        def _():
            my = lax.axis_index("i")
            left = (my - 1) % N_DEV
            right = (my + 1) % N_DEV

            barrier_sem = pltpu.get_barrier_semaphore()
            for nbr in (left, right):
                pl.semaphore_signal(
                    barrier_sem, inc=1,
                    device_id=(nbr,), device_id_type=pl.DeviceIdType.MESH,
                )
            pl.semaphore_wait(barrier_sem, 2)

            comm_ref[0, :, :] = out_ref[:, :]

            for hop in range(N_DEV - 1):
                rdma = pltpu.make_async_remote_copy(
                    src_ref=comm_ref.at[hop],
                    dst_ref=comm_ref.at[hop + 1],
                    send_sem=send_sems.at[hop],
                    recv_sem=recv_sems.at[hop],
                    device_id=(right,),
                    device_id_type=pl.DeviceIdType.MESH,
                )
                rdma.start()
                rdma.wait()
                out_ref[:, :] += comm_ref[hop + 1, :, :]

    return pl.pallas_call(
        body,
        grid=(N_CHUNKS,),
        out_shape=jax.ShapeDtypeStruct((b, d), jnp.float32),
        in_specs=[
            pl.BlockSpec((b, d), lambda k: (0, 0)),
            pl.BlockSpec((d, chunk), lambda k: (0, k)),
            pl.BlockSpec((chunk, d), lambda k: (k, 0)),
        ],
        out_specs=pl.BlockSpec((b, d), lambda k: (0, 0)),
        scratch_shapes=[
            pltpu.VMEM((N_DEV, b, d), jnp.float32),
            pltpu.SemaphoreType.DMA((N_DEV - 1,)),
            pltpu.SemaphoreType.DMA((N_DEV - 1,)),
        ],
        compiler_params=pltpu.CompilerParams(
            collective_id=collective_id,
            vmem_limit_bytes=60 * 1024 * 1024,
        ),
    )(x, win, wout)


def kernel(x, Win0, Wout0, Win1, Wout1, Win2, Wout2):
    x = _layer(x, Win0, Wout0, collective_id=0)
    x = _layer(x, Win1, Wout1, collective_id=1)
    x = _layer(x, Win2, Wout2, collective_id=2)
    return x


# device time: 76714 ns/iter; 1.9171x vs baseline; 1.9171x over previous
import jax
import jax.numpy as jnp
from jax import lax
from jax.experimental import pallas as pl
from jax.experimental.pallas import tpu as pltpu

N_DEV = 4
N_LAYERS = 3
NB = 4


def kernel(x, Win0, Wout0, Win1, Wout1, Win2, Wout2):
    b, d = x.shape
    h_per = Win0.shape[1]
    cb = d // NB

    def body(x_ref, w0_ref, wo0_ref, w1_ref, wo1_ref, w2_ref, wo2_ref,
             out_ref, h_ref, wbuf1, wbuf2, send_p, recv_p,
             w1_sems, w2_sems, s_sems, r_sems):
        wins = [w0_ref, w1_ref, w2_ref]
        wouts = [wo0_ref, wo1_ref, wo2_ref]

        my = lax.axis_index("i")
        peers = (
            jnp.bitwise_xor(my, 1),
            3 - my,
            jnp.bitwise_xor(my, 2),
        )

        def copy_win(l, c, slot):
            return pltpu.make_async_copy(
                wins[l].at[pl.ds(c * cb, cb), :], wbuf1.at[slot],
                w1_sems.at[slot],
            )

        def copy_wout(l, j, slot):
            return pltpu.make_async_copy(
                wouts[l].at[:, pl.ds(j * cb, cb)], wbuf2.at[slot],
                w2_sems.at[slot],
            )

        def p_rdma(l, j, di):
            return pltpu.make_async_remote_copy(
                src_ref=send_p.at[l, j], dst_ref=recv_p.at[l, j, di],
                send_sem=s_sems.at[l, j, di], recv_sem=r_sems.at[l, j, di],
                device_id=(peers[di],), device_id_type=pl.DeviceIdType.MESH,
            )

        def reduced_block(l, j):
            for di in range(3):
                p_rdma(l, j, di).wait_recv()
            return (send_p[l, j] + recv_p[l, j, 0]
                    + recv_p[l, j, 1] + recv_p[l, j, 2])

        barrier_sem = pltpu.get_barrier_semaphore()
        for di in range(3):
            pl.semaphore_signal(
                barrier_sem, inc=1,
                device_id=(peers[di],), device_id_type=pl.DeviceIdType.MESH,
            )
        pl.semaphore_wait(barrier_sem, 3)

        copy_win(0, 0, 0).start()
        copy_win(0, 1, 1).start()

        def issue_next(l, phase, step):
            nxt = step + 2
            if phase == 1:
                if nxt < NB:
                    copy_win(l, nxt, step % 2).start()
                else:
                    copy_wout(l, nxt - NB, nxt - NB).start()
            else:
                if nxt < NB:
                    copy_wout(l, nxt, step % 2).start()
                elif l + 1 < N_LAYERS:
                    copy_win(l + 1, nxt - NB, nxt - NB).start()

        for l in range(N_LAYERS):
            for c in range(NB):
                if l == 0:
                    xblk = x_ref[:, pl.ds(c * cb, cb)]
                else:
                    xblk = reduced_block(l - 1, c)
                copy_win(l, c, c % 2).wait()
                hc = jnp.dot(xblk, wbuf1[c % 2],
                             preferred_element_type=jnp.float32)
                if c == 0:
                    h_ref[:, :] = hc
                else:
                    h_ref[:, :] += hc
                issue_next(l, 1, c)
            h_ref[:, :] = jnp.maximum(h_ref[:, :], 0.0)

            for j in range(NB):
                copy_wout(l, j, j % 2).wait()
                send_p[l, j] = jnp.dot(h_ref[:, :], wbuf2[j % 2],
                                       preferred_element_type=jnp.float32)
                for di in range(3):
                    p_rdma(l, j, di).start()
                issue_next(l, 2, j)

        last = N_LAYERS - 1
        for j in range(NB):
            out_ref[:, pl.ds(j * cb, cb)] = reduced_block(last, j)

        for l in range(N_LAYERS):
            for j in range(NB):
                for di in range(3):
                    p_rdma(l, j, di).wait_send()

    any_spec = pl.BlockSpec(memory_space=pl.ANY)
    vmem_spec = pl.BlockSpec(memory_space=pltpu.VMEM)
    return pl.pallas_call(
        body,
        out_shape=jax.ShapeDtypeStruct((b, d), jnp.float32),
        in_specs=[vmem_spec] + [any_spec] * 6,
        out_specs=vmem_spec,
        scratch_shapes=[
            pltpu.VMEM((b, h_per), jnp.float32),
            pltpu.VMEM((2, cb, h_per), jnp.float32),
            pltpu.VMEM((2, h_per, cb), jnp.float32),
            pltpu.VMEM((N_LAYERS, NB, b, cb), jnp.float32),
            pltpu.VMEM((N_LAYERS, NB, 3, b, cb), jnp.float32),
            pltpu.SemaphoreType.DMA((2,)),
            pltpu.SemaphoreType.DMA((2,)),
            pltpu.SemaphoreType.DMA((N_LAYERS, NB, 3)),
            pltpu.SemaphoreType.DMA((N_LAYERS, NB, 3)),
        ],
        compiler_params=pltpu.CompilerParams(
            collective_id=0,
            vmem_limit_bytes=60 * 1024 * 1024,
        ),
    )(x, Win0, Wout0, Win1, Wout1, Win2, Wout2)
